# Initial kernel scaffold; baseline (speedup 1.0000x reference)
import jax
import jax.numpy as jnp
from jax.experimental import pallas as pl


def kernel(x, x_ex, DFS, STATUS, edge_index, batch, x_ex_mean, x_ex_std, W_lin1, b_lin1, W_lin2, b_lin2, W_emb, b_emb, W_fc2, b_fc2, W_fcall, b_fcall, W_fc3, b_fc3, g1_Wz, g1_Uz, g1_bz, g1_Wr, g1_Ur, g1_br, g1_Wh, g1_Uh, g1_bh, g2_Wz, g2_Uz, g2_bz, g2_Wr, g2_Ur, g2_br, g2_Wh, g2_Uh, g2_bh):
    raise NotImplementedError("write your pallas kernel here")


# speedup vs baseline: 1.0050x; 1.0050x over previous
import functools

import jax
import jax.numpy as jnp
from jax import lax
from jax.experimental import pallas as pl
from jax.experimental.pallas import tpu as pltpu

_R = 2000


_SELU_SCALE = 1.0507009873554805
_SELU_ALPHA = 1.6732632423543772


def _selu(v):
    neg = _SELU_ALPHA * (jnp.exp(jnp.minimum(v, 0.0)) - 1.0)
    return _SELU_SCALE * jnp.where(v > 0, v, neg)


def _dot(a, b):
    return jnp.dot(a, b, preferred_element_type=jnp.float32)


def _lin_selu_body(x_ref, w_ref, b_ref, o_ref):
    o_ref[...] = _selu(_dot(x_ref[...], w_ref[...]) + b_ref[...])


def _lin_selu(x, W, b2d):
    n, f = x.shape
    fo = W.shape[1]
    return pl.pallas_call(
        _lin_selu_body,
        grid=(n // _R,),
        in_specs=[
            pl.BlockSpec((_R, f), lambda i: (i, 0)),
            pl.BlockSpec((f, fo), lambda i: (0, 0)),
            pl.BlockSpec((1, fo), lambda i: (0, 0)),
        ],
        out_specs=pl.BlockSpec((_R, fo), lambda i: (i, 0)),
        out_shape=jax.ShapeDtypeStruct((n, fo), jnp.float32),
    )(x, W, b2d)


def _gru_math(x, m, Wz, Uz, bz, Wr, Ur, br, Wh, Uh, bh, W2, b2):
    z = jax.nn.sigmoid(_dot(m, Wz[...]) + _dot(x, Uz[...]) + bz[...])
    r = jax.nn.sigmoid(_dot(m, Wr[...]) + _dot(x, Ur[...]) + br[...])
    h = jnp.tanh(_dot(m, Wh[...]) + _dot(r * x, Uh[...]) + bh[...])
    x2 = (1.0 - z) * x + z * h
    return _selu(_dot(x2, W2[...]) + b2[...])


def _gru_body(x_ref, p0_ref, p1_ref, Wz, Uz, bz, Wr, Ur, br, Wh, Uh, bh,
              W2, b2, o_ref):
    m = p0_ref[...] + p1_ref[...]
    o_ref[...] = _gru_math(x_ref[...], m, Wz, Uz, bz, Wr, Ur, br, Wh, Uh, bh,
                           W2, b2)


def _w_specs(h):
    sq = pl.BlockSpec((h, h), lambda i: (0, 0))
    row = pl.BlockSpec((1, h), lambda i: (0, 0))
    return [sq, sq, row, sq, sq, row, sq, sq, row]


def _gru_layer(x, p0, p1, g, W2, b2):
    n, h = x.shape
    blk = pl.BlockSpec((_R, h), lambda i: (i, 0))
    return pl.pallas_call(
        _gru_body,
        grid=(n // _R,),
        in_specs=[blk, blk, blk] + _w_specs(h) + [
            pl.BlockSpec((h, h), lambda i: (0, 0)),
            pl.BlockSpec((1, h), lambda i: (0, 0)),
        ],
        out_specs=blk,
        out_shape=jax.ShapeDtypeStruct((n, h), jnp.float32),
    )(x, p0, p1, *g, W2, b2)


def _gru_pool_body(x_ref, p0_ref, p1_ref, batch_ref, Wz, Uz, bz, Wr, Ur, br,
                   Wh, Uh, bh, W2, b2, sums_ref, counts_ref, *, nb):
    i = pl.program_id(0)
    m = p0_ref[...] + p1_ref[...]
    x4 = _gru_math(x_ref[...], m, Wz, Uz, bz, Wr, Ur, br, Wh, Uh, bh, W2, b2)
    b = batch_ref[0, 0, :]
    onehot = (lax.broadcasted_iota(jnp.int32, (nb, _R), 0)
              == b[None, :]).astype(jnp.float32)
    s = _dot(onehot, x4)
    c = jnp.sum(onehot, axis=1, keepdims=True)

    @pl.when(i == 0)
    def _init():
        sums_ref[...] = jnp.zeros_like(sums_ref)
        counts_ref[...] = jnp.zeros_like(counts_ref)

    sums_ref[...] += s
    counts_ref[...] += c


def _gru_pool_layer(x, p0, p1, batch3, nb, g, W2, b2):
    n, h = x.shape
    blk = pl.BlockSpec((_R, h), lambda i: (i, 0))
    return pl.pallas_call(
        functools.partial(_gru_pool_body, nb=nb),
        grid=(n // _R,),
        in_specs=[blk, blk, blk,
                  pl.BlockSpec((1, 1, _R), lambda i: (i, 0, 0))]
                 + _w_specs(h) + [
            pl.BlockSpec((h, h), lambda i: (0, 0)),
            pl.BlockSpec((1, h), lambda i: (0, 0)),
        ],
        out_specs=[pl.BlockSpec((nb, h), lambda i: (0, 0)),
                   pl.BlockSpec((nb, 1), lambda i: (0, 0))],
        out_shape=[jax.ShapeDtypeStruct((nb, h), jnp.float32),
                   jax.ShapeDtypeStruct((nb, 1), jnp.float32)],
    )(x, p0, p1, batch3, *g, W2, b2)


def _l2n(v):
    return v / jnp.maximum(jnp.sqrt(jnp.sum(v * v, axis=1, keepdims=True)),
                           1e-12)


def _head_body(sums_ref, counts_ref, xe_ref, mean_ref, std_ref, W_fc2, b_fc2,
               W_emb, b_emb, W_fcall, b_fcall, W_fc3, b_fc3,
               out_ref, feat_ref):
    x5 = sums_ref[...] / jnp.maximum(counts_ref[...], 1.0)
    feat = _selu(_dot(x5, W_fc2[...]) + b_fc2[...])
    xe = (xe_ref[...] - mean_ref[...]) / std_ref[...]
    emb = _l2n(_selu(_dot(xe, W_emb[...]) + b_emb[...]))
    xa = _l2n(jnp.concatenate([feat, emb], axis=1))
    xa = _selu(_dot(xa, W_fcall[...]) + b_fcall[...])
    out_ref[...] = _dot(xa, W_fc3[...]) + b_fc3[...]
    feat_ref[...] = feat


def _head(sums, counts, xe_sel, mean_sel, std_sel, W_fc2, b_fc2, W_emb, b_emb,
          W_fcall, b_fcall, W_fc3, b_fc3):
    nb = sums.shape[0]
    full = lambda a: pl.BlockSpec(a.shape, lambda: tuple(0 for _ in a.shape))
    args = (sums, counts, xe_sel, mean_sel, std_sel, W_fc2, b_fc2, W_emb,
            b_emb, W_fcall, b_fcall, W_fc3, b_fc3)
    return pl.pallas_call(
        _head_body,
        in_specs=[full(a) for a in args],
        out_specs=[pl.BlockSpec((nb, 1), lambda: (0, 0)),
                   pl.BlockSpec((nb, 16), lambda: (0, 0))],
        out_shape=[jax.ShapeDtypeStruct((nb, 1), jnp.float32),
                   jax.ShapeDtypeStruct((nb, 16), jnp.float32)],
    )(*args)


def kernel(x, x_ex, DFS, STATUS, edge_index, batch, x_ex_mean, x_ex_std,
           W_lin1, b_lin1, W_lin2, b_lin2, W_emb, b_emb, W_fc2, b_fc2,
           W_fcall, b_fcall, W_fc3, b_fc3,
           g1_Wz, g1_Uz, g1_bz, g1_Wr, g1_Ur, g1_br, g1_Wh, g1_Uh, g1_bh,
           g2_Wz, g2_Uz, g2_bz, g2_Wr, g2_Ur, g2_br, g2_Wh, g2_Uh, g2_bh):
    n, fin = x.shape
    nb = DFS.shape[0]
    row = lambda v: v.reshape(1, -1)
    g1 = (g1_Wz, g1_Uz, row(g1_bz), g1_Wr, g1_Ur, row(g1_br),
          g1_Wh, g1_Uh, row(g1_bh))
    g2 = (g2_Wz, g2_Uz, row(g2_bz), g2_Wr, g2_Ur, row(g2_br),
          g2_Wh, g2_Uh, row(g2_bh))

    src = edge_index[0]
    dst = edge_index[1]
    batch3 = batch.reshape(n // _R, 1, _R)
    zeros_n = jnp.zeros((n, W_lin1.shape[1]), jnp.float32)

    x1 = _lin_selu(x, W_lin1, row(b_lin1))
    m1 = zeros_n.at[dst].add(x1[src])
    x3 = _gru_layer(x1, m1, zeros_n, g1, W_lin2, row(b_lin2))
    m2 = zeros_n.at[dst].add(x3[src])
    sums, counts = _gru_pool_layer(x3, m2, zeros_n, batch3, nb, g2,
                                   W_lin2, row(b_lin2))

    xe_sel = jnp.concatenate([x_ex[:, :7], x_ex[:, 8:9]], axis=1)
    mean_sel = jnp.concatenate([x_ex_mean[:7], x_ex_mean[8:9]]).reshape(1, -1)
    std_sel = jnp.concatenate([x_ex_std[:7], x_ex_std[8:9]]).reshape(1, -1)

    x_out, x_features = _head(sums, counts, xe_sel, mean_sel, std_sel,
                              W_fc2, row(b_fc2), W_emb, row(b_emb),
                              W_fcall, row(b_fcall), W_fc3, row(b_fc3))
    return (x_out, x_features)


# speedup vs baseline: 14.0312x; 13.9616x over previous
import functools

import jax
import jax.numpy as jnp
from jax import lax
from jax.experimental import pallas as pl
from jax.experimental.pallas import tpu as pltpu
from jax.experimental.pallas import tpu_sc as plsc

_R = 2000

_NC = 2
_NS = 16
_NW = _NC * _NS
_CH = 100
_KB = 8


def _sc_scatter_body(n, cpt, x_hbm, src_hbm, dst_hbm, zero_hbm, out_hbm,
                     sidx, didx, rows, m_sh, gsem):
    cid = lax.axis_index("c")
    sid = lax.axis_index("s")
    widg = cid * _NS + sid

    @pl.when(sid == 0)
    def _zero():
        pltpu.sync_copy(zero_hbm, m_sh)

    plsc.subcore_barrier()

    def outer(ib, _):
        pltpu.sync_copy(src_hbm.at[widg, pl.ds(ib * _KB, _KB)], sidx)
        pltpu.sync_copy(dst_hbm.at[widg, pl.ds(ib * _KB, _KB)], didx)
        for j in range(_KB):
            pltpu.async_copy(x_hbm.at[sidx.at[j]], rows, gsem).wait()
            pltpu.sync_copy(rows, m_sh.at[didx.at[j]], add=True)
        return 0

    lax.fori_loop(0, cpt // _KB, outer, 0)
    plsc.subcore_barrier()

    @pl.when(sid == 0)
    def _writeback():
        pltpu.sync_copy(m_sh, out_hbm.at[cid])


def _sc_message(x1, src3, dst3, zero_n):
    n, h = x1.shape
    cpt = src3.shape[1]
    mesh = plsc.VectorSubcoreMesh(core_axis_name="c", subcore_axis_name="s",
                                  num_cores=_NC, num_subcores=_NS)
    body = functools.partial(_sc_scatter_body, n, cpt)
    return pl.kernel(
        body,
        out_type=jax.ShapeDtypeStruct((_NC, n, h), jnp.float32),
        mesh=mesh,
        scratch_types=[
            pltpu.VMEM((_KB, _CH), jnp.int32),
            pltpu.VMEM((_KB, _CH), jnp.int32),
            pltpu.VMEM((_CH, h), jnp.float32),
            pltpu.VMEM_SHARED((n, h), jnp.float32),
            pltpu.SemaphoreType.DMA,
        ],
        compiler_params=pltpu.CompilerParams(use_tc_tiling_on_sc=False),
    )(x1, src3, dst3, zero_n)


_SELU_SCALE = 1.0507009873554805
_SELU_ALPHA = 1.6732632423543772


def _selu(v):
    neg = _SELU_ALPHA * (jnp.exp(jnp.minimum(v, 0.0)) - 1.0)
    return _SELU_SCALE * jnp.where(v > 0, v, neg)


def _dot(a, b):
    return jnp.dot(a, b, preferred_element_type=jnp.float32)


def _lin_selu_body(x_ref, w_ref, b_ref, o_ref):
    o_ref[...] = _selu(_dot(x_ref[...], w_ref[...]) + b_ref[...])


def _lin_selu(x, W, b2d):
    n, f = x.shape
    fo = W.shape[1]
    return pl.pallas_call(
        _lin_selu_body,
        grid=(n // _R,),
        in_specs=[
            pl.BlockSpec((_R, f), lambda i: (i, 0)),
            pl.BlockSpec((f, fo), lambda i: (0, 0)),
            pl.BlockSpec((1, fo), lambda i: (0, 0)),
        ],
        out_specs=pl.BlockSpec((_R, fo), lambda i: (i, 0)),
        out_shape=jax.ShapeDtypeStruct((n, fo), jnp.float32),
    )(x, W, b2d)


def _gru_math(x, m, Wz, Uz, bz, Wr, Ur, br, Wh, Uh, bh, W2, b2):
    z = jax.nn.sigmoid(_dot(m, Wz[...]) + _dot(x, Uz[...]) + bz[...])
    r = jax.nn.sigmoid(_dot(m, Wr[...]) + _dot(x, Ur[...]) + br[...])
    h = jnp.tanh(_dot(m, Wh[...]) + _dot(r * x, Uh[...]) + bh[...])
    x2 = (1.0 - z) * x + z * h
    return _selu(_dot(x2, W2[...]) + b2[...])


def _gru_body(x_ref, p0_ref, p1_ref, Wz, Uz, bz, Wr, Ur, br, Wh, Uh, bh,
              W2, b2, o_ref):
    m = p0_ref[...] + p1_ref[...]
    o_ref[...] = _gru_math(x_ref[...], m, Wz, Uz, bz, Wr, Ur, br, Wh, Uh, bh,
                           W2, b2)


def _w_specs(h):
    sq = pl.BlockSpec((h, h), lambda i: (0, 0))
    row = pl.BlockSpec((1, h), lambda i: (0, 0))
    return [sq, sq, row, sq, sq, row, sq, sq, row]


def _gru_layer(x, p0, p1, g, W2, b2):
    n, h = x.shape
    blk = pl.BlockSpec((_R, h), lambda i: (i, 0))
    return pl.pallas_call(
        _gru_body,
        grid=(n // _R,),
        in_specs=[blk, blk, blk] + _w_specs(h) + [
            pl.BlockSpec((h, h), lambda i: (0, 0)),
            pl.BlockSpec((1, h), lambda i: (0, 0)),
        ],
        out_specs=blk,
        out_shape=jax.ShapeDtypeStruct((n, h), jnp.float32),
    )(x, p0, p1, *g, W2, b2)


def _gru_pool_body(x_ref, p0_ref, p1_ref, batch_ref, Wz, Uz, bz, Wr, Ur, br,
                   Wh, Uh, bh, W2, b2, sums_ref, counts_ref, *, nb):
    i = pl.program_id(0)
    m = p0_ref[...] + p1_ref[...]
    x4 = _gru_math(x_ref[...], m, Wz, Uz, bz, Wr, Ur, br, Wh, Uh, bh, W2, b2)
    b = batch_ref[0, 0, :]
    onehot = (lax.broadcasted_iota(jnp.int32, (nb, _R), 0)
              == b[None, :]).astype(jnp.float32)
    s = _dot(onehot, x4)
    c = jnp.sum(onehot, axis=1, keepdims=True)

    @pl.when(i == 0)
    def _init():
        sums_ref[...] = jnp.zeros_like(sums_ref)
        counts_ref[...] = jnp.zeros_like(counts_ref)

    sums_ref[...] += s
    counts_ref[...] += c


def _gru_pool_layer(x, p0, p1, batch3, nb, g, W2, b2):
    n, h = x.shape
    blk = pl.BlockSpec((_R, h), lambda i: (i, 0))
    return pl.pallas_call(
        functools.partial(_gru_pool_body, nb=nb),
        grid=(n // _R,),
        in_specs=[blk, blk, blk,
                  pl.BlockSpec((1, 1, _R), lambda i: (i, 0, 0))]
                 + _w_specs(h) + [
            pl.BlockSpec((h, h), lambda i: (0, 0)),
            pl.BlockSpec((1, h), lambda i: (0, 0)),
        ],
        out_specs=[pl.BlockSpec((nb, h), lambda i: (0, 0)),
                   pl.BlockSpec((nb, 1), lambda i: (0, 0))],
        out_shape=[jax.ShapeDtypeStruct((nb, h), jnp.float32),
                   jax.ShapeDtypeStruct((nb, 1), jnp.float32)],
    )(x, p0, p1, batch3, *g, W2, b2)


def _l2n(v):
    return v / jnp.maximum(jnp.sqrt(jnp.sum(v * v, axis=1, keepdims=True)),
                           1e-12)


def _head_body(sums_ref, counts_ref, xe_ref, mean_ref, std_ref, W_fc2, b_fc2,
               W_emb, b_emb, W_fcall, b_fcall, W_fc3, b_fc3,
               out_ref, feat_ref):
    x5 = sums_ref[...] / jnp.maximum(counts_ref[...], 1.0)
    feat = _selu(_dot(x5, W_fc2[...]) + b_fc2[...])
    xe = (xe_ref[...] - mean_ref[...]) / std_ref[...]
    emb = _l2n(_selu(_dot(xe, W_emb[...]) + b_emb[...]))
    xa = _l2n(jnp.concatenate([feat, emb], axis=1))
    xa = _selu(_dot(xa, W_fcall[...]) + b_fcall[...])
    out_ref[...] = _dot(xa, W_fc3[...]) + b_fc3[...]
    feat_ref[...] = feat


def _head(sums, counts, xe_sel, mean_sel, std_sel, W_fc2, b_fc2, W_emb, b_emb,
          W_fcall, b_fcall, W_fc3, b_fc3):
    nb = sums.shape[0]
    full = lambda a: pl.BlockSpec(a.shape, lambda: tuple(0 for _ in a.shape))
    args = (sums, counts, xe_sel, mean_sel, std_sel, W_fc2, b_fc2, W_emb,
            b_emb, W_fcall, b_fcall, W_fc3, b_fc3)
    return pl.pallas_call(
        _head_body,
        in_specs=[full(a) for a in args],
        out_specs=[pl.BlockSpec((nb, 1), lambda: (0, 0)),
                   pl.BlockSpec((nb, 16), lambda: (0, 0))],
        out_shape=[jax.ShapeDtypeStruct((nb, 1), jnp.float32),
                   jax.ShapeDtypeStruct((nb, 16), jnp.float32)],
    )(*args)


def kernel(x, x_ex, DFS, STATUS, edge_index, batch, x_ex_mean, x_ex_std,
           W_lin1, b_lin1, W_lin2, b_lin2, W_emb, b_emb, W_fc2, b_fc2,
           W_fcall, b_fcall, W_fc3, b_fc3,
           g1_Wz, g1_Uz, g1_bz, g1_Wr, g1_Ur, g1_br, g1_Wh, g1_Uh, g1_bh,
           g2_Wz, g2_Uz, g2_bz, g2_Wr, g2_Ur, g2_br, g2_Wh, g2_Uh, g2_bh):
    n, fin = x.shape
    nb = DFS.shape[0]
    row = lambda v: v.reshape(1, -1)
    g1 = (g1_Wz, g1_Uz, row(g1_bz), g1_Wr, g1_Ur, row(g1_br),
          g1_Wh, g1_Uh, row(g1_bh))
    g2 = (g2_Wz, g2_Uz, row(g2_bz), g2_Wr, g2_Ur, row(g2_br),
          g2_Wh, g2_Uh, row(g2_bh))

    e = edge_index.shape[1]
    cpt = e // (_NW * _CH)
    src3 = edge_index[0].reshape(_NW, cpt, _CH)
    dst3 = edge_index[1].reshape(_NW, cpt, _CH)
    batch3 = batch.reshape(n // _R, 1, _R)
    zeros_n = jnp.zeros((n, W_lin1.shape[1]), jnp.float32)

    x1 = _lin_selu(x, W_lin1, row(b_lin1))
    m1 = _sc_message(x1, src3, dst3, zeros_n)
    x3 = _gru_layer(x1, m1[0], m1[1], g1, W_lin2, row(b_lin2))
    m2 = _sc_message(x3, src3, dst3, zeros_n)
    sums, counts = _gru_pool_layer(x3, m2[0], m2[1], batch3, nb, g2,
                                   W_lin2, row(b_lin2))

    xe_sel = jnp.concatenate([x_ex[:, :7], x_ex[:, 8:9]], axis=1)
    mean_sel = jnp.concatenate([x_ex_mean[:7], x_ex_mean[8:9]]).reshape(1, -1)
    std_sel = jnp.concatenate([x_ex_std[:7], x_ex_std[8:9]]).reshape(1, -1)

    x_out, x_features = _head(sums, counts, xe_sel, mean_sel, std_sel,
                              W_fc2, row(b_fc2), W_emb, row(b_emb),
                              W_fcall, row(b_fcall), W_fc3, row(b_fc3))
    return (x_out, x_features)


# speedup vs baseline: 30.0279x; 2.1401x over previous
import functools

import jax
import jax.numpy as jnp
from jax import lax
from jax.experimental import pallas as pl
from jax.experimental.pallas import tpu as pltpu
from jax.experimental.pallas import tpu_sc as plsc

_R = 2000

_NC = 2
_NS = 16
_NW = _NC * _NS
_CH = 100
_KB = 20


def _sc_scatter_body(n, cpt, x_hbm, src_hbm, dst_hbm, zero_hbm, out_hbm,
                     sidx, didx, rows, m_sh, gsem, ssem, isem):
    cid = lax.axis_index("c")
    sid = lax.axis_index("s")
    widg = cid * _NS + sid
    ng = cpt // _KB

    @pl.when(sid == 0)
    def _zero():
        pltpu.sync_copy(zero_hbm, m_sh)

    plsc.subcore_barrier()

    pltpu.sync_copy(src_hbm.at[widg, pl.ds(0, _KB)], sidx.at[0])
    pltpu.sync_copy(dst_hbm.at[widg, pl.ds(0, _KB)], didx.at[0])

    def outer(g, _):
        p = lax.rem(g, 2)
        q = 1 - p
        nxt = lax.rem(g + 1, ng)
        ipre = [
            pltpu.async_copy(src_hbm.at[widg, pl.ds(nxt * _KB, _KB)],
                             sidx.at[q], isem),
            pltpu.async_copy(dst_hbm.at[widg, pl.ds(nxt * _KB, _KB)],
                             didx.at[q], isem),
        ]
        gds = [pltpu.async_copy(x_hbm.at[sidx.at[p, j]], rows.at[j], gsem)
               for j in range(_KB)]
        for d in gds:
            d.wait()
        sds = [pltpu.async_copy(rows.at[j], m_sh.at[didx.at[p, j]], ssem,
                                add=True)
               for j in range(_KB)]
        for d in sds:
            d.wait()
        for d in ipre:
            d.wait()
        return 0

    lax.fori_loop(0, ng, outer, 0)
    plsc.subcore_barrier()

    @pl.when(sid == 0)
    def _writeback():
        pltpu.sync_copy(m_sh, out_hbm.at[cid])


def _sc_message(x1, src3, dst3, zero_n):
    n, h = x1.shape
    cpt = src3.shape[1]
    mesh = plsc.VectorSubcoreMesh(core_axis_name="c", subcore_axis_name="s",
                                  num_cores=_NC, num_subcores=_NS)
    body = functools.partial(_sc_scatter_body, n, cpt)
    return pl.kernel(
        body,
        out_type=jax.ShapeDtypeStruct((_NC, n, h), jnp.float32),
        mesh=mesh,
        scratch_types=[
            pltpu.VMEM((2, _KB, _CH), jnp.int32),
            pltpu.VMEM((2, _KB, _CH), jnp.int32),
            pltpu.VMEM((_KB, _CH, h), jnp.float32),
            pltpu.VMEM_SHARED((n, h), jnp.float32),
            pltpu.SemaphoreType.DMA,
            pltpu.SemaphoreType.DMA,
            pltpu.SemaphoreType.DMA,
        ],
        compiler_params=pltpu.CompilerParams(use_tc_tiling_on_sc=False),
    )(x1, src3, dst3, zero_n)


_SELU_SCALE = 1.0507009873554805
_SELU_ALPHA = 1.6732632423543772


def _selu(v):
    neg = _SELU_ALPHA * (jnp.exp(jnp.minimum(v, 0.0)) - 1.0)
    return _SELU_SCALE * jnp.where(v > 0, v, neg)


def _dot(a, b):
    return jnp.dot(a, b, preferred_element_type=jnp.float32)


def _lin_selu_body(x_ref, w_ref, b_ref, o_ref):
    o_ref[...] = _selu(_dot(x_ref[...], w_ref[...]) + b_ref[...])


def _lin_selu(x, W, b2d):
    n, f = x.shape
    fo = W.shape[1]
    return pl.pallas_call(
        _lin_selu_body,
        grid=(n // _R,),
        in_specs=[
            pl.BlockSpec((_R, f), lambda i: (i, 0)),
            pl.BlockSpec((f, fo), lambda i: (0, 0)),
            pl.BlockSpec((1, fo), lambda i: (0, 0)),
        ],
        out_specs=pl.BlockSpec((_R, fo), lambda i: (i, 0)),
        out_shape=jax.ShapeDtypeStruct((n, fo), jnp.float32),
    )(x, W, b2d)


def _gru_math(x, m, Wz, Uz, bz, Wr, Ur, br, Wh, Uh, bh, W2, b2):
    z = jax.nn.sigmoid(_dot(m, Wz[...]) + _dot(x, Uz[...]) + bz[...])
    r = jax.nn.sigmoid(_dot(m, Wr[...]) + _dot(x, Ur[...]) + br[...])
    h = jnp.tanh(_dot(m, Wh[...]) + _dot(r * x, Uh[...]) + bh[...])
    x2 = (1.0 - z) * x + z * h
    return _selu(_dot(x2, W2[...]) + b2[...])


def _gru_body(x_ref, p0_ref, p1_ref, Wz, Uz, bz, Wr, Ur, br, Wh, Uh, bh,
              W2, b2, o_ref):
    m = p0_ref[...] + p1_ref[...]
    o_ref[...] = _gru_math(x_ref[...], m, Wz, Uz, bz, Wr, Ur, br, Wh, Uh, bh,
                           W2, b2)


def _w_specs(h):
    sq = pl.BlockSpec((h, h), lambda i: (0, 0))
    row = pl.BlockSpec((1, h), lambda i: (0, 0))
    return [sq, sq, row, sq, sq, row, sq, sq, row]


def _gru_layer(x, p0, p1, g, W2, b2):
    n, h = x.shape
    blk = pl.BlockSpec((_R, h), lambda i: (i, 0))
    return pl.pallas_call(
        _gru_body,
        grid=(n // _R,),
        in_specs=[blk, blk, blk] + _w_specs(h) + [
            pl.BlockSpec((h, h), lambda i: (0, 0)),
            pl.BlockSpec((1, h), lambda i: (0, 0)),
        ],
        out_specs=blk,
        out_shape=jax.ShapeDtypeStruct((n, h), jnp.float32),
    )(x, p0, p1, *g, W2, b2)


def _gru_pool_body(x_ref, p0_ref, p1_ref, batch_ref, Wz, Uz, bz, Wr, Ur, br,
                   Wh, Uh, bh, W2, b2, sums_ref, counts_ref, *, nb):
    i = pl.program_id(0)
    m = p0_ref[...] + p1_ref[...]
    x4 = _gru_math(x_ref[...], m, Wz, Uz, bz, Wr, Ur, br, Wh, Uh, bh, W2, b2)
    b = batch_ref[0, 0, :]
    onehot = (lax.broadcasted_iota(jnp.int32, (nb, _R), 0)
              == b[None, :]).astype(jnp.float32)
    s = _dot(onehot, x4)
    c = jnp.sum(onehot, axis=1, keepdims=True)

    @pl.when(i == 0)
    def _init():
        sums_ref[...] = jnp.zeros_like(sums_ref)
        counts_ref[...] = jnp.zeros_like(counts_ref)

    sums_ref[...] += s
    counts_ref[...] += c


def _gru_pool_layer(x, p0, p1, batch3, nb, g, W2, b2):
    n, h = x.shape
    blk = pl.BlockSpec((_R, h), lambda i: (i, 0))
    return pl.pallas_call(
        functools.partial(_gru_pool_body, nb=nb),
        grid=(n // _R,),
        in_specs=[blk, blk, blk,
                  pl.BlockSpec((1, 1, _R), lambda i: (i, 0, 0))]
                 + _w_specs(h) + [
            pl.BlockSpec((h, h), lambda i: (0, 0)),
            pl.BlockSpec((1, h), lambda i: (0, 0)),
        ],
        out_specs=[pl.BlockSpec((nb, h), lambda i: (0, 0)),
                   pl.BlockSpec((nb, 1), lambda i: (0, 0))],
        out_shape=[jax.ShapeDtypeStruct((nb, h), jnp.float32),
                   jax.ShapeDtypeStruct((nb, 1), jnp.float32)],
    )(x, p0, p1, batch3, *g, W2, b2)


def _l2n(v):
    return v / jnp.maximum(jnp.sqrt(jnp.sum(v * v, axis=1, keepdims=True)),
                           1e-12)


def _head_body(sums_ref, counts_ref, xe_ref, mean_ref, std_ref, W_fc2, b_fc2,
               W_emb, b_emb, W_fcall, b_fcall, W_fc3, b_fc3,
               out_ref, feat_ref):
    x5 = sums_ref[...] / jnp.maximum(counts_ref[...], 1.0)
    feat = _selu(_dot(x5, W_fc2[...]) + b_fc2[...])
    xe = (xe_ref[...] - mean_ref[...]) / std_ref[...]
    emb = _l2n(_selu(_dot(xe, W_emb[...]) + b_emb[...]))
    xa = _l2n(jnp.concatenate([feat, emb], axis=1))
    xa = _selu(_dot(xa, W_fcall[...]) + b_fcall[...])
    out_ref[...] = _dot(xa, W_fc3[...]) + b_fc3[...]
    feat_ref[...] = feat


def _head(sums, counts, xe_sel, mean_sel, std_sel, W_fc2, b_fc2, W_emb, b_emb,
          W_fcall, b_fcall, W_fc3, b_fc3):
    nb = sums.shape[0]
    full = lambda a: pl.BlockSpec(a.shape, lambda: tuple(0 for _ in a.shape))
    args = (sums, counts, xe_sel, mean_sel, std_sel, W_fc2, b_fc2, W_emb,
            b_emb, W_fcall, b_fcall, W_fc3, b_fc3)
    return pl.pallas_call(
        _head_body,
        in_specs=[full(a) for a in args],
        out_specs=[pl.BlockSpec((nb, 1), lambda: (0, 0)),
                   pl.BlockSpec((nb, 16), lambda: (0, 0))],
        out_shape=[jax.ShapeDtypeStruct((nb, 1), jnp.float32),
                   jax.ShapeDtypeStruct((nb, 16), jnp.float32)],
    )(*args)


def kernel(x, x_ex, DFS, STATUS, edge_index, batch, x_ex_mean, x_ex_std,
           W_lin1, b_lin1, W_lin2, b_lin2, W_emb, b_emb, W_fc2, b_fc2,
           W_fcall, b_fcall, W_fc3, b_fc3,
           g1_Wz, g1_Uz, g1_bz, g1_Wr, g1_Ur, g1_br, g1_Wh, g1_Uh, g1_bh,
           g2_Wz, g2_Uz, g2_bz, g2_Wr, g2_Ur, g2_br, g2_Wh, g2_Uh, g2_bh):
    n, fin = x.shape
    nb = DFS.shape[0]
    row = lambda v: v.reshape(1, -1)
    g1 = (g1_Wz, g1_Uz, row(g1_bz), g1_Wr, g1_Ur, row(g1_br),
          g1_Wh, g1_Uh, row(g1_bh))
    g2 = (g2_Wz, g2_Uz, row(g2_bz), g2_Wr, g2_Ur, row(g2_br),
          g2_Wh, g2_Uh, row(g2_bh))

    e = edge_index.shape[1]
    cpt = e // (_NW * _CH)
    src3 = edge_index[0].reshape(_NW, cpt, _CH)
    dst3 = edge_index[1].reshape(_NW, cpt, _CH)
    batch3 = batch.reshape(n // _R, 1, _R)
    zeros_n = jnp.zeros((n, W_lin1.shape[1]), jnp.float32)

    x1 = _lin_selu(x, W_lin1, row(b_lin1))
    m1 = _sc_message(x1, src3, dst3, zeros_n)
    x3 = _gru_layer(x1, m1[0], m1[1], g1, W_lin2, row(b_lin2))
    m2 = _sc_message(x3, src3, dst3, zeros_n)
    sums, counts = _gru_pool_layer(x3, m2[0], m2[1], batch3, nb, g2,
                                   W_lin2, row(b_lin2))

    xe_sel = jnp.concatenate([x_ex[:, :7], x_ex[:, 8:9]], axis=1)
    mean_sel = jnp.concatenate([x_ex_mean[:7], x_ex_mean[8:9]]).reshape(1, -1)
    std_sel = jnp.concatenate([x_ex_std[:7], x_ex_std[8:9]]).reshape(1, -1)

    x_out, x_features = _head(sums, counts, xe_sel, mean_sel, std_sel,
                              W_fc2, row(b_fc2), W_emb, row(b_emb),
                              W_fcall, row(b_fcall), W_fc3, row(b_fc3))
    return (x_out, x_features)


# speedup vs baseline: 31.9441x; 1.0638x over previous
import functools

import jax
import jax.numpy as jnp
from jax import lax
from jax.experimental import pallas as pl
from jax.experimental.pallas import tpu as pltpu
from jax.experimental.pallas import tpu_sc as plsc

_NC = 2
_NS = 16
_NW = _NC * _NS
_CH = 100
_KB = 20

_G = 16
_RP = 128

_SELU_SCALE = 1.0507009873554805
_SELU_ALPHA = 1.6732632423543772


def _selu(v):
    neg = _SELU_ALPHA * (jnp.exp(jnp.minimum(v, 0.0)) - 1.0)
    return _SELU_SCALE * jnp.where(v > 0, v, neg)


def _dot(a, b):
    return jnp.dot(a, b, preferred_element_type=jnp.float32)


def _sc_scatter_body(n, cpt, x_hbm, src_hbm, dst_hbm, zero_hbm, out_hbm,
                     sidx, didx, rows, m_sh, gsem, ssem, isem):
    cid = lax.axis_index("c")
    sid = lax.axis_index("s")
    widg = cid * _NS + sid
    ng = cpt // _KB

    @pl.when(sid == 0)
    def _zero():
        pltpu.sync_copy(zero_hbm, m_sh)

    plsc.subcore_barrier()

    pltpu.sync_copy(src_hbm.at[widg, pl.ds(0, _KB)], sidx.at[0])
    pltpu.sync_copy(dst_hbm.at[widg, pl.ds(0, _KB)], didx.at[0])

    def outer(g, _):
        p = lax.rem(g, 2)
        q = 1 - p
        nxt = lax.rem(g + 1, ng)
        ipre = [
            pltpu.async_copy(src_hbm.at[widg, pl.ds(nxt * _KB, _KB)],
                             sidx.at[q], isem),
            pltpu.async_copy(dst_hbm.at[widg, pl.ds(nxt * _KB, _KB)],
                             didx.at[q], isem),
        ]
        gds = [pltpu.async_copy(x_hbm.at[sidx.at[p, j]], rows.at[j], gsem)
               for j in range(_KB)]
        for d in gds:
            d.wait()
        sds = [pltpu.async_copy(rows.at[j], m_sh.at[didx.at[p, j]], ssem,
                                add=True)
               for j in range(_KB)]
        for d in sds:
            d.wait()
        for d in ipre:
            d.wait()
        return 0

    lax.fori_loop(0, ng, outer, 0)
    plsc.subcore_barrier()

    @pl.when(sid == 0)
    def _writeback():
        pltpu.sync_copy(m_sh, out_hbm.at[cid])


def _sc_message(x_lin, src3, dst3, zero_n):
    n, h = x_lin.shape
    cpt = src3.shape[1]
    mesh = plsc.VectorSubcoreMesh(core_axis_name="c", subcore_axis_name="s",
                                  num_cores=_NC, num_subcores=_NS)
    body = functools.partial(_sc_scatter_body, n, cpt)
    return pl.kernel(
        body,
        out_type=jax.ShapeDtypeStruct((_NC, n, h), jnp.float32),
        mesh=mesh,
        scratch_types=[
            pltpu.VMEM((2, _KB, _CH), jnp.int32),
            pltpu.VMEM((2, _KB, _CH), jnp.int32),
            pltpu.VMEM((_KB, _CH, h), jnp.float32),
            pltpu.VMEM_SHARED((n, h), jnp.float32),
            pltpu.SemaphoreType.DMA,
            pltpu.SemaphoreType.DMA,
            pltpu.SemaphoreType.DMA,
        ],
        compiler_params=pltpu.CompilerParams(use_tc_tiling_on_sc=False),
    )(x_lin, src3, dst3, zero_n)


def _lin_selu_body(x_ref, w_ref, b_ref, o_ref):
    o_ref[...] = _selu(_dot(x_ref[...], w_ref[...]) + b_ref[...])


def _lin_selu(x, W, b2d):
    n, f = x.shape
    fo = W.shape[1]
    return pl.pallas_call(
        _lin_selu_body,
        grid=(n // _RP,),
        in_specs=[
            pl.BlockSpec((_RP, f), lambda i: (i, 0)),
            pl.BlockSpec((f, fo), lambda i: (0, 0)),
            pl.BlockSpec((1, fo), lambda i: (0, 0)),
        ],
        out_specs=pl.BlockSpec((_RP, fo), lambda i: (i, 0)),
        out_shape=jax.ShapeDtypeStruct((n, fo), jnp.float32),
    )(x, W, b2d)


def _gru_math(x, m, Wz, Uz, bz, Wr, Ur, br, Wh, Uh, bh, W2, b2):
    z = jax.nn.sigmoid(_dot(m, Wz[...]) + _dot(x, Uz[...]) + bz[...])
    r = jax.nn.sigmoid(_dot(m, Wr[...]) + _dot(x, Ur[...]) + br[...])
    h = jnp.tanh(_dot(m, Wh[...]) + _dot(r * x, Uh[...]) + bh[...])
    x2 = (1.0 - z) * x + z * h
    return _selu(_dot(x2, W2[...]) + b2[...])


def _gru_body(x_ref, p0_ref, p1_ref, Wz, Uz, bz, Wr, Ur, br, Wh, Uh, bh,
              W2, b2, o_ref):
    m = p0_ref[...] + p1_ref[...]
    o_ref[...] = _gru_math(x_ref[...], m, Wz, Uz, bz, Wr, Ur, br, Wh, Uh, bh,
                           W2, b2)


def _w_specs():
    sq = pl.BlockSpec((128, 128), lambda i: (0, 0))
    row = pl.BlockSpec((1, 128), lambda i: (0, 0))
    return [sq, sq, row, sq, sq, row, sq, sq, row]


def _gru_layer(x, p0, p1, g, W2, b2):
    n, _ = x.shape
    blk = pl.BlockSpec((_RP, 128), lambda i: (i, 0))
    return pl.pallas_call(
        _gru_body,
        grid=(n // _RP,),
        in_specs=[blk, blk, blk] + _w_specs() + [
            pl.BlockSpec((128, 128), lambda i: (0, 0)),
            pl.BlockSpec((1, 128), lambda i: (0, 0)),
        ],
        out_specs=blk,
        out_shape=jax.ShapeDtypeStruct((n, 128), jnp.float32),
    )(x, p0, p1, *g, W2, b2)


def _gru_pool_body(x_ref, p0_ref, p1_ref, batch_ref, Wz, Uz, bz, Wr, Ur, br,
                   Wh, Uh, bh, W2, b2, sums_ref, counts_ref, *, nb):
    i = pl.program_id(0)
    m = p0_ref[...] + p1_ref[...]
    x4 = _gru_math(x_ref[...], m, Wz, Uz, bz, Wr, Ur, br, Wh, Uh, bh, W2, b2)
    iota_b = lax.broadcasted_iota(jnp.int32, (nb, _RP), 0)
    ri = lax.broadcasted_iota(jnp.int32, (128, 8), 0)
    ci = lax.broadcasted_iota(jnp.int32, (128, 8), 1)
    s = jnp.zeros((nb, 8), jnp.float32)
    c = jnp.zeros((nb, 1), jnp.float32)
    for g in range(_G):
        bg = batch_ref[0, g, :]
        oh = (iota_b == bg[None, :]).astype(jnp.float32)
        sel = (ri == g * 8 + ci).astype(jnp.float32)
        s = s + _dot(oh, _dot(x4, sel))
        c = c + jnp.sum(oh, axis=1, keepdims=True)

    @pl.when(i == 0)
    def _init():
        sums_ref[...] = jnp.zeros_like(sums_ref)
        counts_ref[...] = jnp.zeros_like(counts_ref)

    sums_ref[...] += s
    counts_ref[...] += c


def _gru_pool_layer(x, p0, p1, batch3, nb, g, W2, b2):
    n, _ = x.shape
    blk = pl.BlockSpec((_RP, 128), lambda i: (i, 0))
    return pl.pallas_call(
        functools.partial(_gru_pool_body, nb=nb),
        grid=(n // _RP,),
        in_specs=[blk, blk, blk,
                  pl.BlockSpec((1, _G, _RP), lambda i: (i, 0, 0))]
                 + _w_specs() + [
            pl.BlockSpec((128, 128), lambda i: (0, 0)),
            pl.BlockSpec((1, 128), lambda i: (0, 0)),
        ],
        out_specs=[pl.BlockSpec((nb, 8), lambda i: (0, 0)),
                   pl.BlockSpec((nb, 1), lambda i: (0, 0))],
        out_shape=[jax.ShapeDtypeStruct((nb, 8), jnp.float32),
                   jax.ShapeDtypeStruct((nb, 1), jnp.float32)],
    )(x, p0, p1, batch3, *g, W2, b2)


def _l2n(v):
    return v / jnp.maximum(jnp.sqrt(jnp.sum(v * v, axis=1, keepdims=True)),
                           1e-12)


def _head_body(sums_ref, counts_ref, xe_ref, mean_ref, std_ref, W_fc2, b_fc2,
               W_emb, b_emb, W_fcall, b_fcall, W_fc3, b_fc3,
               out_ref, feat_ref):
    x5 = sums_ref[...] / jnp.maximum(counts_ref[...], 1.0)
    feat = _selu(_dot(x5, W_fc2[...]) + b_fc2[...])
    xe = (xe_ref[...] - mean_ref[...]) / std_ref[...]
    emb = _l2n(_selu(_dot(xe, W_emb[...]) + b_emb[...]))
    xa = _l2n(jnp.concatenate([feat, emb], axis=1))
    xa = _selu(_dot(xa, W_fcall[...]) + b_fcall[...])
    out_ref[...] = _dot(xa, W_fc3[...]) + b_fc3[...]
    feat_ref[...] = feat


def _head(sums, counts, xe_sel, mean_sel, std_sel, W_fc2, b_fc2, W_emb, b_emb,
          W_fcall, b_fcall, W_fc3, b_fc3):
    nb = sums.shape[0]
    full = lambda a: pl.BlockSpec(a.shape, lambda: tuple(0 for _ in a.shape))
    args = (sums, counts, xe_sel, mean_sel, std_sel, W_fc2, b_fc2, W_emb,
            b_emb, W_fcall, b_fcall, W_fc3, b_fc3)
    return pl.pallas_call(
        _head_body,
        in_specs=[full(a) for a in args],
        out_specs=[pl.BlockSpec((nb, 1), lambda: (0, 0)),
                   pl.BlockSpec((nb, 16), lambda: (0, 0))],
        out_shape=[jax.ShapeDtypeStruct((nb, 1), jnp.float32),
                   jax.ShapeDtypeStruct((nb, 16), jnp.float32)],
    )(*args)


def kernel(x, x_ex, DFS, STATUS, edge_index, batch, x_ex_mean, x_ex_std,
           W_lin1, b_lin1, W_lin2, b_lin2, W_emb, b_emb, W_fc2, b_fc2,
           W_fcall, b_fcall, W_fc3, b_fc3,
           g1_Wz, g1_Uz, g1_bz, g1_Wr, g1_Ur, g1_br, g1_Wh, g1_Uh, g1_bh,
           g2_Wz, g2_Uz, g2_bz, g2_Wr, g2_Ur, g2_br, g2_Wh, g2_Uh, g2_bh):
    n, fin = x.shape
    nb = DFS.shape[0]
    npad = ((n + _G * _RP - 1) // (_G * _RP)) * (_G * _RP)
    eye = jnp.eye(_G, dtype=jnp.float32)
    bd = lambda W: jnp.kron(eye, W)
    tb = lambda b: jnp.tile(b, _G).reshape(1, 128)
    g1 = (bd(g1_Wz), bd(g1_Uz), tb(g1_bz), bd(g1_Wr), bd(g1_Ur), tb(g1_br),
          bd(g1_Wh), bd(g1_Uh), tb(g1_bh))
    g2 = (bd(g2_Wz), bd(g2_Uz), tb(g2_bz), bd(g2_Wr), bd(g2_Ur), tb(g2_br),
          bd(g2_Wh), bd(g2_Uh), tb(g2_bh))
    W1bd, b1t = bd(W_lin1), tb(b_lin1)
    W2bd, b2t = bd(W_lin2), tb(b_lin2)

    e = edge_index.shape[1]
    cpt = e // (_NW * _CH)
    src3 = edge_index[0].reshape(_NW, cpt, _CH)
    dst3 = edge_index[1].reshape(_NW, cpt, _CH)
    zero_n = jnp.zeros((npad, fin), jnp.float32)

    xpk = jnp.pad(x, ((0, npad - n), (0, 0))).reshape(npad // _G, 128)
    batchp = jnp.pad(batch, (0, npad - n), constant_values=nb)
    batch3 = batchp.reshape(npad // (_G * _RP), _RP, _G).transpose(0, 2, 1)

    x1 = _lin_selu(xpk, W1bd, b1t)
    m1 = _sc_message(x1.reshape(npad, fin), src3, dst3, zero_n)
    m1pk = m1.reshape(_NC, npad // _G, 128)
    x3 = _gru_layer(x1, m1pk[0], m1pk[1], g1, W2bd, b2t)
    m2 = _sc_message(x3.reshape(npad, fin), src3, dst3, zero_n)
    m2pk = m2.reshape(_NC, npad // _G, 128)
    sums, counts = _gru_pool_layer(x3, m2pk[0], m2pk[1], batch3, nb, g2,
                                   W2bd, b2t)

    xe_sel = jnp.concatenate([x_ex[:, :7], x_ex[:, 8:9]], axis=1)
    mean_sel = jnp.concatenate([x_ex_mean[:7], x_ex_mean[8:9]]).reshape(1, -1)
    std_sel = jnp.concatenate([x_ex_std[:7], x_ex_std[8:9]]).reshape(1, -1)

    row = lambda v: v.reshape(1, -1)
    x_out, x_features = _head(sums, counts, xe_sel, mean_sel, std_sel,
                              W_fc2, row(b_fc2), W_emb, row(b_emb),
                              W_fcall, row(b_fcall), W_fc3, row(b_fc3))
    return (x_out, x_features)


# speedup vs baseline: 33.5956x; 1.0517x over previous
import functools

import jax
import jax.numpy as jnp
from jax import lax
from jax.experimental import pallas as pl
from jax.experimental.pallas import tpu as pltpu
from jax.experimental.pallas import tpu_sc as plsc

_NC = 2
_NS = 16
_NW = _NC * _NS
_CH = 100
_KB = 20

_G = 16
_RP = 128

_SELU_SCALE = 1.0507009873554805
_SELU_ALPHA = 1.6732632423543772


def _selu(v):
    neg = _SELU_ALPHA * (jnp.exp(jnp.minimum(v, 0.0)) - 1.0)
    return _SELU_SCALE * jnp.where(v > 0, v, neg)


def _dot(a, b):
    return jnp.dot(a, b, preferred_element_type=jnp.float32)


def _sc_scatter_body(n, cpt, x_hbm, ei_hbm, zero_hbm, out_hbm,
                     sidx, didx, rows, m_sh, x_sh, gsem, ssem, isem):
    cid = lax.axis_index("c")
    sid = lax.axis_index("s")
    widg = cid * _NS + sid
    ng = cpt // _KB

    @pl.when(sid == 0)
    def _zero():
        pltpu.sync_copy(zero_hbm, m_sh)

    @pl.when(sid == 1)
    def _stage():
        pltpu.sync_copy(x_hbm, x_sh)

    plsc.subcore_barrier()

    pltpu.sync_copy(ei_hbm.at[0, widg, pl.ds(0, _KB)], sidx.at[0])
    pltpu.sync_copy(ei_hbm.at[1, widg, pl.ds(0, _KB)], didx.at[0])

    def outer(g, _):
        p = lax.rem(g, 2)
        q = 1 - p
        nxt = lax.rem(g + 1, ng)
        ipre = [
            pltpu.async_copy(ei_hbm.at[0, widg, pl.ds(nxt * _KB, _KB)],
                             sidx.at[q], isem),
            pltpu.async_copy(ei_hbm.at[1, widg, pl.ds(nxt * _KB, _KB)],
                             didx.at[q], isem),
        ]
        gds = [pltpu.async_copy(x_sh.at[sidx.at[p, j]], rows.at[j], gsem)
               for j in range(_KB)]
        for d in gds:
            d.wait()
        sds = [pltpu.async_copy(rows.at[j], m_sh.at[didx.at[p, j]], ssem,
                                add=True)
               for j in range(_KB)]
        for d in sds:
            d.wait()
        for d in ipre:
            d.wait()
        return 0

    lax.fori_loop(0, ng, outer, 0)
    plsc.subcore_barrier()

    @pl.when(sid == 0)
    def _writeback():
        pltpu.sync_copy(m_sh, out_hbm.at[cid])


def _sc_message(x_lin, ei4, zero_n):
    n, h = x_lin.shape
    cpt = ei4.shape[2]
    mesh = plsc.VectorSubcoreMesh(core_axis_name="c", subcore_axis_name="s",
                                  num_cores=_NC, num_subcores=_NS)
    body = functools.partial(_sc_scatter_body, n, cpt)
    return pl.kernel(
        body,
        out_type=jax.ShapeDtypeStruct((_NC, n, h), jnp.float32),
        mesh=mesh,
        scratch_types=[
            pltpu.VMEM((2, _KB, _CH), jnp.int32),
            pltpu.VMEM((2, _KB, _CH), jnp.int32),
            pltpu.VMEM((_KB, _CH, h), jnp.float32),
            pltpu.VMEM_SHARED((n, h), jnp.float32),
            pltpu.VMEM_SHARED((n, h), jnp.float32),
            pltpu.SemaphoreType.DMA,
            pltpu.SemaphoreType.DMA,
            pltpu.SemaphoreType.DMA,
        ],
        compiler_params=pltpu.CompilerParams(use_tc_tiling_on_sc=False),
    )(x_lin, ei4, zero_n)


def _lin_selu_body(x_ref, w_ref, b_ref, o_ref):
    o_ref[...] = _selu(_dot(x_ref[...], w_ref[...]) + b_ref[...])


def _lin_selu(x, W, b2d):
    n, f = x.shape
    fo = W.shape[1]
    return pl.pallas_call(
        _lin_selu_body,
        grid=(n // _RP,),
        in_specs=[
            pl.BlockSpec((_RP, f), lambda i: (i, 0)),
            pl.BlockSpec((f, fo), lambda i: (0, 0)),
            pl.BlockSpec((1, fo), lambda i: (0, 0)),
        ],
        out_specs=pl.BlockSpec((_RP, fo), lambda i: (i, 0)),
        out_shape=jax.ShapeDtypeStruct((n, fo), jnp.float32),
    )(x, W, b2d)


def _gru_math(x, m, Wz, Uz, bz, Wr, Ur, br, Wh, Uh, bh, W2, b2):
    z = jax.nn.sigmoid(_dot(m, Wz[...]) + _dot(x, Uz[...]) + bz[...])
    r = jax.nn.sigmoid(_dot(m, Wr[...]) + _dot(x, Ur[...]) + br[...])
    h = jnp.tanh(_dot(m, Wh[...]) + _dot(r * x, Uh[...]) + bh[...])
    x2 = (1.0 - z) * x + z * h
    return _selu(_dot(x2, W2[...]) + b2[...])


def _gru_body(x_ref, p0_ref, p1_ref, Wz, Uz, bz, Wr, Ur, br, Wh, Uh, bh,
              W2, b2, o_ref):
    m = p0_ref[...] + p1_ref[...]
    o_ref[...] = _gru_math(x_ref[...], m, Wz, Uz, bz, Wr, Ur, br, Wh, Uh, bh,
                           W2, b2)


def _w_specs():
    sq = pl.BlockSpec((128, 128), lambda i: (0, 0))
    row = pl.BlockSpec((1, 128), lambda i: (0, 0))
    return [sq, sq, row, sq, sq, row, sq, sq, row]


def _gru_layer(x, m2d, g, W2, b2):
    n, _ = x.shape
    nblk = n // _RP
    blk = pl.BlockSpec((_RP, 128), lambda i: (i, 0))
    p1blk = pl.BlockSpec((_RP, 128), lambda i: (i + nblk, 0))
    return pl.pallas_call(
        _gru_body,
        grid=(nblk,),
        in_specs=[blk, blk, p1blk] + _w_specs() + [
            pl.BlockSpec((128, 128), lambda i: (0, 0)),
            pl.BlockSpec((1, 128), lambda i: (0, 0)),
        ],
        out_specs=blk,
        out_shape=jax.ShapeDtypeStruct((n, 128), jnp.float32),
    )(x, m2d, m2d, *g, W2, b2)


def _gru_pool_body(x_ref, p0_ref, p1_ref, batch_ref, Wz, Uz, bz, Wr, Ur, br,
                   Wh, Uh, bh, W2, b2, sums_ref, counts_ref, *, nb):
    i = pl.program_id(0)
    m = p0_ref[...] + p1_ref[...]
    x4 = _gru_math(x_ref[...], m, Wz, Uz, bz, Wr, Ur, br, Wh, Uh, bh, W2, b2)
    iota_b = lax.broadcasted_iota(jnp.int32, (nb, _RP), 0)
    ri = lax.broadcasted_iota(jnp.int32, (128, 8), 0)
    ci = lax.broadcasted_iota(jnp.int32, (128, 8), 1)
    s = jnp.zeros((nb, 8), jnp.float32)
    c = jnp.zeros((nb, 1), jnp.float32)
    for g in range(_G):
        bg = batch_ref[0, g, :]
        oh = (iota_b == bg[None, :]).astype(jnp.float32)
        sel = (ri == g * 8 + ci).astype(jnp.float32)
        s = s + _dot(oh, _dot(x4, sel))
        c = c + jnp.sum(oh, axis=1, keepdims=True)

    @pl.when(i == 0)
    def _init():
        sums_ref[...] = jnp.zeros_like(sums_ref)
        counts_ref[...] = jnp.zeros_like(counts_ref)

    sums_ref[...] += s
    counts_ref[...] += c


def _gru_pool_layer(x, m2d, batch3, nb, g, W2, b2):
    n, _ = x.shape
    nblk = n // _RP
    blk = pl.BlockSpec((_RP, 128), lambda i: (i, 0))
    p1blk = pl.BlockSpec((_RP, 128), lambda i: (i + nblk, 0))
    return pl.pallas_call(
        functools.partial(_gru_pool_body, nb=nb),
        grid=(nblk,),
        in_specs=[blk, blk, p1blk,
                  pl.BlockSpec((1, _G, _RP), lambda i: (i, 0, 0))]
                 + _w_specs() + [
            pl.BlockSpec((128, 128), lambda i: (0, 0)),
            pl.BlockSpec((1, 128), lambda i: (0, 0)),
        ],
        out_specs=[pl.BlockSpec((nb, 8), lambda i: (0, 0)),
                   pl.BlockSpec((nb, 1), lambda i: (0, 0))],
        out_shape=[jax.ShapeDtypeStruct((nb, 8), jnp.float32),
                   jax.ShapeDtypeStruct((nb, 1), jnp.float32)],
    )(x, m2d, m2d, batch3, *g, W2, b2)


def _l2n(v):
    return v / jnp.maximum(jnp.sqrt(jnp.sum(v * v, axis=1, keepdims=True)),
                           1e-12)


def _head_body(sums_ref, counts_ref, xe_ref, mean_ref, std_ref, W_fc2, b_fc2,
               W_emb, b_emb, W_fcall, b_fcall, W_fc3, b_fc3,
               out_ref, feat_ref):
    x5 = sums_ref[...] / jnp.maximum(counts_ref[...], 1.0)
    feat = _selu(_dot(x5, W_fc2[...]) + b_fc2[...])
    xe = (xe_ref[...] - mean_ref[...]) / std_ref[...]
    emb = _l2n(_selu(_dot(xe, W_emb[...]) + b_emb[...]))
    xa = _l2n(jnp.concatenate([feat, emb], axis=1))
    xa = _selu(_dot(xa, W_fcall[...]) + b_fcall[...])
    out_ref[...] = _dot(xa, W_fc3[...]) + b_fc3[...]
    feat_ref[...] = feat


def _head(sums, counts, xe_sel, mean_sel, std_sel, W_fc2, b_fc2, W_emb, b_emb,
          W_fcall, b_fcall, W_fc3, b_fc3):
    nb = sums.shape[0]
    full = lambda a: pl.BlockSpec(a.shape, lambda: tuple(0 for _ in a.shape))
    args = (sums, counts, xe_sel, mean_sel, std_sel, W_fc2, b_fc2, W_emb,
            b_emb, W_fcall, b_fcall, W_fc3, b_fc3)
    return pl.pallas_call(
        _head_body,
        in_specs=[full(a) for a in args],
        out_specs=[pl.BlockSpec((nb, 1), lambda: (0, 0)),
                   pl.BlockSpec((nb, 16), lambda: (0, 0))],
        out_shape=[jax.ShapeDtypeStruct((nb, 1), jnp.float32),
                   jax.ShapeDtypeStruct((nb, 16), jnp.float32)],
    )(*args)


def kernel(x, x_ex, DFS, STATUS, edge_index, batch, x_ex_mean, x_ex_std,
           W_lin1, b_lin1, W_lin2, b_lin2, W_emb, b_emb, W_fc2, b_fc2,
           W_fcall, b_fcall, W_fc3, b_fc3,
           g1_Wz, g1_Uz, g1_bz, g1_Wr, g1_Ur, g1_br, g1_Wh, g1_Uh, g1_bh,
           g2_Wz, g2_Uz, g2_bz, g2_Wr, g2_Ur, g2_br, g2_Wh, g2_Uh, g2_bh):
    n, fin = x.shape
    nb = DFS.shape[0]
    npad = ((n + _G * _RP - 1) // (_G * _RP)) * (_G * _RP)
    eye = jnp.eye(_G, dtype=jnp.float32)
    bd = lambda W: jnp.kron(eye, W)
    tb = lambda b: jnp.tile(b, _G).reshape(1, 128)
    g1 = (bd(g1_Wz), bd(g1_Uz), tb(g1_bz), bd(g1_Wr), bd(g1_Ur), tb(g1_br),
          bd(g1_Wh), bd(g1_Uh), tb(g1_bh))
    g2 = (bd(g2_Wz), bd(g2_Uz), tb(g2_bz), bd(g2_Wr), bd(g2_Ur), tb(g2_br),
          bd(g2_Wh), bd(g2_Uh), tb(g2_bh))
    W1bd, b1t = bd(W_lin1), tb(b_lin1)
    W2bd, b2t = bd(W_lin2), tb(b_lin2)

    e = edge_index.shape[1]
    cpt = e // (_NW * _CH)
    ei4 = edge_index.reshape(2, _NW, cpt, _CH)
    zero_n = jnp.zeros((npad, fin), jnp.float32)

    xpk = jnp.pad(x, ((0, npad - n), (0, 0))).reshape(npad // _G, 128)
    batchp = jnp.pad(batch, (0, npad - n), constant_values=nb)
    batch3 = batchp.reshape(npad // (_G * _RP), _RP, _G).transpose(0, 2, 1)

    x1 = _lin_selu(xpk, W1bd, b1t)
    m1 = _sc_message(x1.reshape(npad, fin), ei4, zero_n)
    x3 = _gru_layer(x1, m1.reshape(_NC * npad // _G, 128), g1, W2bd, b2t)
    m2 = _sc_message(x3.reshape(npad, fin), ei4, zero_n)
    sums, counts = _gru_pool_layer(x3, m2.reshape(_NC * npad // _G, 128),
                                   batch3, nb, g2, W2bd, b2t)

    xe_sel = jnp.concatenate([x_ex[:, :7], x_ex[:, 8:9]], axis=1)
    mean_sel = jnp.concatenate([x_ex_mean[:7], x_ex_mean[8:9]]).reshape(1, -1)
    std_sel = jnp.concatenate([x_ex_std[:7], x_ex_std[8:9]]).reshape(1, -1)

    row = lambda v: v.reshape(1, -1)
    x_out, x_features = _head(sums, counts, xe_sel, mean_sel, std_sel,
                              W_fc2, row(b_fc2), W_emb, row(b_emb),
                              W_fcall, row(b_fcall), W_fc3, row(b_fc3))
    return (x_out, x_features)
